# baseline (speedup 1.0000x reference)
import functools

import jax
import jax.numpy as jnp
from jax import lax
from jax.experimental import pallas as pl
from jax.experimental.pallas import tpu as pltpu
from jax.experimental.pallas import tpu_sc as plsc

NC = 2
NS = 16
NW = NC * NS
L = 16
C = 96


def _make_spmm(n, d, ch):
    nps = 8 * (n // (8 * NS))
    nrem = n - nps * NS
    nd16 = d // L
    mesh = plsc.VectorSubcoreMesh(
        core_axis_name="c", subcore_axis_name="s", num_cores=NC,
        num_subcores=NS)

    @functools.partial(
        pl.kernel,
        out_type=jax.ShapeDtypeStruct((NC, n, d), jnp.float32),
        mesh=mesh,
        scratch_types=[
            pltpu.VMEM((6, 2, C), jnp.int32),
            pltpu.VMEM((6, C), jnp.float32),
            pltpu.VMEM((C, d), jnp.float32),
            pltpu.VMEM((C, d), jnp.float32),
            pltpu.VMEM((C, d), jnp.float32),
            pltpu.VMEM_SHARED((n, d), jnp.float32),
            pltpu.SemaphoreType.DMA,
            pltpu.SemaphoreType.DMA,
            pltpu.SemaphoreType.DMA,
        ],
    )
    def spmm(x_hbm, eidx_hbm, w_hbm, out_hbm, ebuf, wbuf, rows0, rows1, rows2,
             acc, gsem, isem, ssem):
        cid = lax.axis_index("c")
        sid = lax.axis_index("s")
        wid = sid * NC + cid
        rowbufs = (rows0, rows1, rows2)

        def idx_start(c, slot):
            pltpu.make_async_copy(eidx_hbm.at[wid, c], ebuf.at[slot],
                                  isem).start()
            pltpu.make_async_copy(w_hbm.at[wid, c], wbuf.at[slot],
                                  isem).start()

        def idx_wait(c, slot):
            pltpu.make_async_copy(eidx_hbm.at[wid, c], ebuf.at[slot],
                                  isem).wait()
            pltpu.make_async_copy(w_hbm.at[wid, c], wbuf.at[slot],
                                  isem).wait()

        def gather(c, slot, rbuf, op):
            getattr(pltpu.make_async_copy(x_hbm.at[ebuf.at[slot, 0]], rbuf,
                                          gsem), op)()

        def scatter_wait(slot, rbuf):
            pltpu.make_async_copy(rbuf, acc.at[ebuf.at[slot, 1]], ssem).wait()

        @pl.loop(0, C)
        def _zero_rows(i):
            z = jnp.zeros((L,), jnp.float32)
            for j in range(nd16):
                rows0[i, pl.ds(j * L, L)] = z

        base = sid * nps
        full, rem = nps // C, nps % C
        for k in range(full):
            pltpu.sync_copy(rows0, acc.at[pl.ds(base + k * C, C)])
        if rem:
            pltpu.sync_copy(rows0.at[pl.ds(0, rem)],
                            acc.at[pl.ds(base + full * C, rem)])
        if nrem:
            @pl.when(sid == NS - 1)
            def _():
                pltpu.sync_copy(rows0.at[pl.ds(0, nrem)],
                                acc.at[pl.ds(nps * NS, nrem)])
        plsc.subcore_barrier()

        pltpu.sync_copy(eidx_hbm.at[wid, 0], ebuf.at[0])
        pltpu.sync_copy(w_hbm.at[wid, 0], wbuf.at[0])
        idx_start(1, 1)
        idx_start(2, 2)
        gather(0, 0, rows0, "start")
        idx_wait(1, 1)
        gather(1, 1, rows1, "start")

        @pl.loop(0, ch, step=12)
        def _chunks(c0):
            for u in range(12):
                c = c0 + u
                rs, es = u % 3, u % 6
                rows = rowbufs[rs]
                gather(c, es, rows, "wait")

                @pl.when(c > 0)
                def _():
                    scatter_wait((u - 1) % 6, rowbufs[(u - 1) % 3])

                @pl.when(c + 2 < ch)
                def _():
                    idx_wait(c + 2, (u + 2) % 6)
                    gather(c + 2, (u + 2) % 6, rowbufs[(u + 2) % 3], "start")

                @pl.loop(0, C // L)
                def _scale(g):
                    wv = wbuf[es, pl.ds(g * L, L)]
                    for l in range(L):
                        wb = lax.gather(
                            wv, jnp.full((L, 1), l, jnp.int32),
                            lax.GatherDimensionNumbers(
                                offset_dims=(), collapsed_slice_dims=(0,),
                                start_index_map=(0,)),
                            slice_sizes=(1,),
                            mode=lax.GatherScatterMode.PROMISE_IN_BOUNDS)
                        e = g * L + l
                        for j in range(nd16):
                            sl = pl.ds(j * L, L)
                            rows[e, sl] = rows[e, sl] * wb

                pltpu.async_copy(rows, acc.at[ebuf.at[es, 1]], ssem,
                                 add=True)

                @pl.when(c + 3 < ch)
                def _():
                    idx_start(c + 3, (u + 3) % 6)

        scatter_wait((ch - 1) % 6, rowbufs[(ch - 1) % 3])

        plsc.subcore_barrier()
        pltpu.sync_copy(acc.at[pl.ds(base, nps)],
                        out_hbm.at[cid, pl.ds(base, nps)])
        if nrem:
            @pl.when(sid == NS - 1)
            def _():
                pltpu.sync_copy(acc.at[pl.ds(nps * NS, nrem)],
                                out_hbm.at[cid, pl.ds(nps * NS, nrem)])

    return spmm


def _row_block(n):
    for rb in (2000, 1000, 400, 200, 40, 8):
        if n % rb == 0:
            return rb
    return n


def _project(x, w1, w2, w3):
    n, d = x.shape
    rb = _row_block(n)

    def body(x_ref, w1_ref, w2_ref, w3_ref, t1_ref, t2_ref, t3_ref):
        xb = x_ref[...]
        t1_ref[...] = jnp.tanh(jnp.dot(xb, w1_ref[...],
                                       preferred_element_type=jnp.float32))
        t2_ref[...] = jnp.tanh(jnp.dot(xb, w2_ref[...],
                                       preferred_element_type=jnp.float32))
        t3_ref[...] = jnp.tanh(jnp.dot(xb, w3_ref[...],
                                       preferred_element_type=jnp.float32))

    out = jax.ShapeDtypeStruct((n, d), jnp.float32)
    wspec = pl.BlockSpec((d, d), lambda i: (0, 0))
    rspec = pl.BlockSpec((rb, d), lambda i: (i, 0))
    return pl.pallas_call(
        body,
        grid=(n // rb,),
        in_specs=[rspec, wspec, wspec, wspec],
        out_specs=[rspec, rspec, rspec],
        out_shape=[out, out, out],
    )(x, w1, w2, w3)


def _combine(p, bias):
    _, n, d = p.shape
    rb = _row_block(n)
    nb = n // rb

    def body(p_ref, b_ref, y_ref, m_ref):
        i = pl.program_id(0)
        y = p_ref[0] + p_ref[1] + b_ref[...]
        y_ref[...] = y
        s = jnp.sum(y, axis=0, keepdims=True)

        @pl.when(i == 0)
        def _():
            m_ref[...] = s

        @pl.when(i > 0)
        def _():
            m_ref[...] = m_ref[...] + s

        @pl.when(i == nb - 1)
        def _():
            m_ref[...] = m_ref[...] * (1.0 / n)

    return pl.pallas_call(
        body,
        grid=(nb,),
        in_specs=[pl.BlockSpec((2, rb, d), lambda i: (0, i, 0)),
                  pl.BlockSpec((1, d), lambda i: (0, 0))],
        out_specs=[pl.BlockSpec((rb, d), lambda i: (i, 0)),
                   pl.BlockSpec((1, d), lambda i: (0, 0))],
        out_shape=[jax.ShapeDtypeStruct((n, d), jnp.float32),
                   jax.ShapeDtypeStruct((1, d), jnp.float32)],
    )(p, bias)


def _final(a1, a2, a3, m, f1wt, f1b, f2wt, f2b):
    n, d = a1.shape
    rb = _row_block(n)

    def body(a1_ref, a2_ref, a3_ref, m_ref, f1w_ref, f1b_ref, f2w_ref,
             f2b_ref, out_ref, fa_ref):
        @pl.when(pl.program_id(0) == 0)
        def _():
            h = jnp.dot(m_ref[...], f1w_ref[...],
                        preferred_element_type=jnp.float32) + f1b_ref[...]
            h = jnp.maximum(h, 0.0)
            logits = jnp.dot(h, f2w_ref[...],
                             preferred_element_type=jnp.float32) + f2b_ref[...]
            mx = jnp.max(logits, axis=0, keepdims=True)
            ex = jnp.exp(logits - mx)
            fa = ex / jnp.sum(ex, axis=0, keepdims=True)
            fa_ref[...] = jnp.concatenate(
                [fa, jnp.zeros((5, d), jnp.float32)], axis=0)

        out_ref[...] = (a1_ref[...] * fa_ref[0:1] +
                        a2_ref[...] * fa_ref[1:2] +
                        a3_ref[...] * fa_ref[2:3])

    rspec = pl.BlockSpec((rb, d), lambda i: (i, 0))
    r = f1wt.shape[1]
    return pl.pallas_call(
        body,
        grid=(n // rb,),
        in_specs=[rspec, rspec, rspec,
                  pl.BlockSpec((3, d), lambda i: (0, 0)),
                  pl.BlockSpec((d, r), lambda i: (0, 0)),
                  pl.BlockSpec((1, r), lambda i: (0, 0)),
                  pl.BlockSpec((r, d), lambda i: (0, 0)),
                  pl.BlockSpec((1, d), lambda i: (0, 0))],
        out_specs=rspec,
        out_shape=jax.ShapeDtypeStruct((n, d), jnp.float32),
        scratch_shapes=[pltpu.VMEM((8, d), jnp.float32)],
    )(a1, a2, a3, m, f1wt, f1b, f2wt, f2b)


def kernel(features, edge_index, edge_weight,
           W1, b1, W2, b2, W3, b3, fc1_w, fc1_b, fc2_w, fc2_b):
    n, d = features.shape
    e = edge_weight.shape[0]
    dst = edge_index[0]
    src = edge_index[1]

    ch = -(-e // (NW * C))
    ch = -12 * (-ch // 12)
    pad = NW * ch * C - e
    srcp = jnp.concatenate([src, jnp.zeros((pad,), jnp.int32)]
                           ).reshape(NW, ch, 1, C)
    dstp = jnp.concatenate([dst, jnp.zeros((pad,), jnp.int32)]
                           ).reshape(NW, ch, 1, C)
    wp = jnp.concatenate([edge_weight, jnp.zeros((pad,), jnp.float32)]
                         ).reshape(NW, ch, C)
    eidx = jnp.concatenate([srcp, dstp], axis=2)

    spmm = _make_spmm(n, d, ch)
    t1, t2, t3 = _project(features, W1, W2, W3)
    zb = jnp.zeros((1, d), jnp.float32)

    u1p = spmm(t1, eidx, wp)
    a1, m1 = _combine(u1p, b1.reshape(1, d))
    u2p = spmm(t2, eidx, wp)
    u2, _ = _combine(u2p, zb)
    v2p = spmm(u2, eidx, wp)
    a2, m2 = _combine(v2p, b2.reshape(1, d))
    u3p = spmm(t3, eidx, wp)
    u3, _ = _combine(u3p, zb)
    v3p = spmm(u3, eidx, wp)
    v3, _ = _combine(v3p, zb)
    w3p = spmm(v3, eidx, wp)
    a3, m3 = _combine(w3p, b3.reshape(1, d))

    m = jnp.concatenate([m1, m2, m3], axis=0)
    return _final(a1, a2, a3, m, fc1_w.T, fc1_b.reshape(1, -1),
                  fc2_w.T, fc2_b.reshape(1, d))


# speedup vs baseline: 1.2566x; 1.2566x over previous
import functools

import jax
import jax.numpy as jnp
from jax import lax
from jax.experimental import pallas as pl
from jax.experimental.pallas import tpu as pltpu
from jax.experimental.pallas import tpu_sc as plsc

NC = 2
NS = 16
NW = NC * NS
L = 16
C = 128
FRAC0 = 0.25


def _make_spmm(n, d, ch0, ch1):
    nps = 8 * (n // (8 * NS))
    nrem = n - nps * NS
    nd16 = d // L
    mesh = plsc.VectorSubcoreMesh(
        core_axis_name="c", subcore_axis_name="s", num_cores=NC,
        num_subcores=NS)

    @functools.partial(
        pl.kernel,
        out_type=jax.ShapeDtypeStruct((NC, n, d), jnp.float32),
        mesh=mesh,
        scratch_types=[
            pltpu.VMEM((4, 2, C), jnp.int32),
            pltpu.VMEM((4, C), jnp.float32),
            pltpu.VMEM((C, d), jnp.float32),
            pltpu.VMEM((C, d), jnp.float32),
            pltpu.VMEM_SHARED((n, d), jnp.float32),
            pltpu.SemaphoreType.DMA,
            pltpu.SemaphoreType.DMA,
            pltpu.SemaphoreType.DMA,
        ],
    )
    def spmm(x_hbm, eidx_hbm, w_hbm, out_hbm, ebuf, wbuf, rows0, rows1, acc,
             gsem, isem, ssem):
        cid = lax.axis_index("c")
        sid = lax.axis_index("s")
        mych = jnp.where(cid == 0, ch0, ch1)
        mybase = jnp.where(cid == 0, sid * ch0, NS * ch0 + sid * ch1)
        rowbufs = (rows0, rows1)

        @pl.loop(0, C)
        def _zero_rows(i):
            z = jnp.zeros((L,), jnp.float32)
            for j in range(nd16):
                rows0[i, pl.ds(j * L, L)] = z

        base = sid * nps
        full, rem = nps // C, nps % C
        for k in range(full):
            pltpu.sync_copy(rows0, acc.at[pl.ds(base + k * C, C)])
        if rem:
            pltpu.sync_copy(rows0.at[pl.ds(0, rem)],
                            acc.at[pl.ds(base + full * C, rem)])
        if nrem:
            @pl.when(sid == NS - 1)
            def _():
                pltpu.sync_copy(rows0.at[pl.ds(0, nrem)],
                                acc.at[pl.ds(nps * NS, nrem)])
        plsc.subcore_barrier()

        pltpu.sync_copy(eidx_hbm.at[mybase], ebuf.at[0])
        pltpu.sync_copy(w_hbm.at[mybase], wbuf.at[0])
        pltpu.make_async_copy(eidx_hbm.at[mybase + 1], ebuf.at[1],
                              isem).start()
        pltpu.make_async_copy(w_hbm.at[mybase + 1], wbuf.at[1], isem).start()
        pltpu.make_async_copy(x_hbm.at[ebuf.at[0, 0]], rows0, gsem).start()

        @pl.loop(0, mych, step=4)
        def _chunks(c0):
            for u in range(4):
                c = c0 + u
                b = u % 2
                nb = 1 - b
                s, sp, s1, s2 = u, (u - 1) % 4, (u + 1) % 4, (u + 2) % 4
                rows = rowbufs[b]
                nxt = rowbufs[nb]
                pltpu.make_async_copy(x_hbm.at[ebuf.at[s, 0]], rows,
                                      gsem).wait()

                @pl.when(c > 0)
                def _():
                    pltpu.make_async_copy(nxt, acc.at[ebuf.at[sp, 1]],
                                          ssem).wait()

                @pl.when(c + 1 < mych)
                def _():
                    pltpu.make_async_copy(
                        eidx_hbm.at[mybase + c + 1], ebuf.at[s1],
                        isem).wait()
                    pltpu.make_async_copy(
                        w_hbm.at[mybase + c + 1], wbuf.at[s1], isem).wait()
                    pltpu.make_async_copy(
                        x_hbm.at[ebuf.at[s1, 0]], nxt, gsem).start()

                @pl.loop(0, C // L)
                def _scale(g):
                    wv = wbuf[s, pl.ds(g * L, L)]
                    for l in range(L):
                        wb = lax.gather(
                            wv, jnp.full((L, 1), l, jnp.int32),
                            lax.GatherDimensionNumbers(
                                offset_dims=(), collapsed_slice_dims=(0,),
                                start_index_map=(0,)),
                            slice_sizes=(1,),
                            mode=lax.GatherScatterMode.PROMISE_IN_BOUNDS)
                        e = g * L + l
                        for j in range(nd16):
                            sl = pl.ds(j * L, L)
                            rows[e, sl] = rows[e, sl] * wb

                pltpu.async_copy(rows, acc.at[ebuf.at[s, 1]], ssem,
                                 add=True)

                @pl.when(c + 2 < mych)
                def _():
                    pltpu.make_async_copy(
                        eidx_hbm.at[mybase + c + 2], ebuf.at[s2],
                        isem).start()
                    pltpu.make_async_copy(
                        w_hbm.at[mybase + c + 2], wbuf.at[s2], isem).start()

        pltpu.make_async_copy(rowbufs[1], acc.at[ebuf.at[3, 1]], ssem).wait()

        plsc.subcore_barrier()
        pltpu.sync_copy(acc.at[pl.ds(base, nps)],
                        out_hbm.at[cid, pl.ds(base, nps)])
        if nrem:
            @pl.when(sid == NS - 1)
            def _():
                pltpu.sync_copy(acc.at[pl.ds(nps * NS, nrem)],
                                out_hbm.at[cid, pl.ds(nps * NS, nrem)])

    return spmm


def _row_block(n):
    for rb in (2000, 1000, 400, 200, 40, 8):
        if n % rb == 0:
            return rb
    return n


def _project(x, w1, w2, w3):
    n, d = x.shape
    rb = _row_block(n)

    def body(x_ref, w1_ref, w2_ref, w3_ref, t1_ref, t2_ref, t3_ref):
        xb = x_ref[...]
        t1_ref[...] = jnp.tanh(jnp.dot(xb, w1_ref[...],
                                       preferred_element_type=jnp.float32))
        t2_ref[...] = jnp.tanh(jnp.dot(xb, w2_ref[...],
                                       preferred_element_type=jnp.float32))
        t3_ref[...] = jnp.tanh(jnp.dot(xb, w3_ref[...],
                                       preferred_element_type=jnp.float32))

    out = jax.ShapeDtypeStruct((n, d), jnp.float32)
    wspec = pl.BlockSpec((d, d), lambda i: (0, 0))
    rspec = pl.BlockSpec((rb, d), lambda i: (i, 0))
    return pl.pallas_call(
        body,
        grid=(n // rb,),
        in_specs=[rspec, wspec, wspec, wspec],
        out_specs=[rspec, rspec, rspec],
        out_shape=[out, out, out],
    )(x, w1, w2, w3)


def _combine(p, bias):
    _, n, d = p.shape
    rb = _row_block(n)
    nb = n // rb

    def body(p_ref, b_ref, y_ref, m_ref):
        i = pl.program_id(0)
        y = p_ref[0] + p_ref[1] + b_ref[...]
        y_ref[...] = y
        s = jnp.sum(y, axis=0, keepdims=True)

        @pl.when(i == 0)
        def _():
            m_ref[...] = s

        @pl.when(i > 0)
        def _():
            m_ref[...] = m_ref[...] + s

        @pl.when(i == nb - 1)
        def _():
            m_ref[...] = m_ref[...] * (1.0 / n)

    return pl.pallas_call(
        body,
        grid=(nb,),
        in_specs=[pl.BlockSpec((2, rb, d), lambda i: (0, i, 0)),
                  pl.BlockSpec((1, d), lambda i: (0, 0))],
        out_specs=[pl.BlockSpec((rb, d), lambda i: (i, 0)),
                   pl.BlockSpec((1, d), lambda i: (0, 0))],
        out_shape=[jax.ShapeDtypeStruct((n, d), jnp.float32),
                   jax.ShapeDtypeStruct((1, d), jnp.float32)],
    )(p, bias)


def _final(a1, a2, a3, m, f1wt, f1b, f2wt, f2b):
    n, d = a1.shape
    rb = _row_block(n)

    def body(a1_ref, a2_ref, a3_ref, m_ref, f1w_ref, f1b_ref, f2w_ref,
             f2b_ref, out_ref, fa_ref):
        @pl.when(pl.program_id(0) == 0)
        def _():
            h = jnp.dot(m_ref[...], f1w_ref[...],
                        preferred_element_type=jnp.float32) + f1b_ref[...]
            h = jnp.maximum(h, 0.0)
            logits = jnp.dot(h, f2w_ref[...],
                             preferred_element_type=jnp.float32) + f2b_ref[...]
            mx = jnp.max(logits, axis=0, keepdims=True)
            ex = jnp.exp(logits - mx)
            fa = ex / jnp.sum(ex, axis=0, keepdims=True)
            fa_ref[...] = jnp.concatenate(
                [fa, jnp.zeros((5, d), jnp.float32)], axis=0)

        out_ref[...] = (a1_ref[...] * fa_ref[0:1] +
                        a2_ref[...] * fa_ref[1:2] +
                        a3_ref[...] * fa_ref[2:3])

    rspec = pl.BlockSpec((rb, d), lambda i: (i, 0))
    r = f1wt.shape[1]
    return pl.pallas_call(
        body,
        grid=(n // rb,),
        in_specs=[rspec, rspec, rspec,
                  pl.BlockSpec((3, d), lambda i: (0, 0)),
                  pl.BlockSpec((d, r), lambda i: (0, 0)),
                  pl.BlockSpec((1, r), lambda i: (0, 0)),
                  pl.BlockSpec((r, d), lambda i: (0, 0)),
                  pl.BlockSpec((1, d), lambda i: (0, 0))],
        out_specs=rspec,
        out_shape=jax.ShapeDtypeStruct((n, d), jnp.float32),
        scratch_shapes=[pltpu.VMEM((8, d), jnp.float32)],
    )(a1, a2, a3, m, f1wt, f1b, f2wt, f2b)


def kernel(features, edge_index, edge_weight,
           W1, b1, W2, b2, W3, b3, fc1_w, fc1_b, fc2_w, fc2_b):
    n, d = features.shape
    e = edge_weight.shape[0]
    dst = edge_index[0]
    src = edge_index[1]

    t = -(-e // C)
    ch0 = max(4, 4 * round(FRAC0 * t / (NS * 4)))
    ch1 = max(4, 4 * (-(-(t - NS * ch0) // (NS * 4))))
    ct = NS * (ch0 + ch1)
    pad = ct * C - e
    srcp = jnp.concatenate([src, jnp.zeros((pad,), jnp.int32)]
                           ).reshape(ct, 1, C)
    dstp = jnp.concatenate([dst, jnp.zeros((pad,), jnp.int32)]
                           ).reshape(ct, 1, C)
    wp = jnp.concatenate([edge_weight, jnp.zeros((pad,), jnp.float32)]
                         ).reshape(ct, C)
    eidx = jnp.concatenate([srcp, dstp], axis=1)

    spmm = _make_spmm(n, d, ch0, ch1)
    t1, t2, t3 = _project(features, W1, W2, W3)
    zb = jnp.zeros((1, d), jnp.float32)

    u1p = spmm(t1, eidx, wp)
    a1, m1 = _combine(u1p, b1.reshape(1, d))
    u2p = spmm(t2, eidx, wp)
    u2, _ = _combine(u2p, zb)
    v2p = spmm(u2, eidx, wp)
    a2, m2 = _combine(v2p, b2.reshape(1, d))
    u3p = spmm(t3, eidx, wp)
    u3, _ = _combine(u3p, zb)
    v3p = spmm(u3, eidx, wp)
    v3, _ = _combine(v3p, zb)
    w3p = spmm(v3, eidx, wp)
    a3, m3 = _combine(w3p, b3.reshape(1, d))

    m = jnp.concatenate([m1, m2, m3], axis=0)
    return _final(a1, a2, a3, m, fc1_w.T, fc1_b.reshape(1, -1),
                  fc2_w.T, fc2_b.reshape(1, d))


# speedup vs baseline: 1.4605x; 1.1622x over previous
import functools

import jax
import jax.numpy as jnp
from jax import lax
from jax.experimental import pallas as pl
from jax.experimental.pallas import tpu as pltpu
from jax.experimental.pallas import tpu_sc as plsc

NC = 2
NS = 16
NW = NC * NS
L = 16
C = 128
FRAC0 = 1.0


def _make_spmm(n, d, ch0, ch1):
    nps = 8 * (n // (8 * NS))
    nrem = n - nps * NS
    nd16 = d // L
    mesh = plsc.VectorSubcoreMesh(
        core_axis_name="c", subcore_axis_name="s", num_cores=NC,
        num_subcores=NS)

    @functools.partial(
        pl.kernel,
        out_type=jax.ShapeDtypeStruct((NC, n, d), jnp.float32),
        mesh=mesh,
        scratch_types=[
            pltpu.VMEM((4, 2, C), jnp.int32),
            pltpu.VMEM((4, C), jnp.float32),
            pltpu.VMEM((C, d), jnp.float32),
            pltpu.VMEM((C, d), jnp.float32),
            pltpu.VMEM_SHARED((n, d), jnp.float32),
            pltpu.SemaphoreType.DMA,
            pltpu.SemaphoreType.DMA,
            pltpu.SemaphoreType.DMA,
        ],
    )
    def spmm(x_hbm, eidx_hbm, w_hbm, out_hbm, ebuf, wbuf, rows0, rows1, acc,
             gsem, isem, ssem):
        cid = lax.axis_index("c")
        sid = lax.axis_index("s")
        mych = jnp.where(cid == 0, ch0, ch1)
        mybase = jnp.where(cid == 0, sid * ch0, NS * ch0 + sid * ch1)
        rowbufs = (rows0, rows1)

        @pl.loop(0, C)
        def _zero_rows(i):
            z = jnp.zeros((L,), jnp.float32)
            for j in range(nd16):
                rows0[i, pl.ds(j * L, L)] = z

        base = sid * nps
        full, rem = nps // C, nps % C
        for k in range(full):
            pltpu.sync_copy(rows0, acc.at[pl.ds(base + k * C, C)])
        if rem:
            pltpu.sync_copy(rows0.at[pl.ds(0, rem)],
                            acc.at[pl.ds(base + full * C, rem)])
        if nrem:
            @pl.when(sid == NS - 1)
            def _():
                pltpu.sync_copy(rows0.at[pl.ds(0, nrem)],
                                acc.at[pl.ds(nps * NS, nrem)])
        plsc.subcore_barrier()

        pltpu.sync_copy(eidx_hbm.at[mybase], ebuf.at[0])
        pltpu.sync_copy(w_hbm.at[mybase], wbuf.at[0])
        pltpu.make_async_copy(eidx_hbm.at[mybase + 1], ebuf.at[1],
                              isem).start()
        pltpu.make_async_copy(w_hbm.at[mybase + 1], wbuf.at[1], isem).start()
        pltpu.make_async_copy(x_hbm.at[ebuf.at[0, 0]], rows0, gsem).start()

        @pl.loop(0, mych, step=4)
        def _chunks(c0):
            for u in range(4):
                c = c0 + u
                b = u % 2
                nb = 1 - b
                s, sp, s1, s2 = u, (u - 1) % 4, (u + 1) % 4, (u + 2) % 4
                rows = rowbufs[b]
                nxt = rowbufs[nb]
                pltpu.make_async_copy(x_hbm.at[ebuf.at[s, 0]], rows,
                                      gsem).wait()

                @pl.when(c > 0)
                def _():
                    pltpu.make_async_copy(nxt, acc.at[ebuf.at[sp, 1]],
                                          ssem).wait()

                @pl.when(c + 1 < mych)
                def _():
                    pltpu.make_async_copy(
                        eidx_hbm.at[mybase + c + 1], ebuf.at[s1],
                        isem).wait()
                    pltpu.make_async_copy(
                        w_hbm.at[mybase + c + 1], wbuf.at[s1], isem).wait()
                    pltpu.make_async_copy(
                        x_hbm.at[ebuf.at[s1, 0]], nxt, gsem).start()

                @pl.loop(0, C // L)
                def _scale(g):
                    wv = wbuf[s, pl.ds(g * L, L)]
                    for l in range(L):
                        wb = lax.gather(
                            wv, jnp.full((L, 1), l, jnp.int32),
                            lax.GatherDimensionNumbers(
                                offset_dims=(), collapsed_slice_dims=(0,),
                                start_index_map=(0,)),
                            slice_sizes=(1,),
                            mode=lax.GatherScatterMode.PROMISE_IN_BOUNDS)
                        e = g * L + l
                        for j in range(nd16):
                            sl = pl.ds(j * L, L)
                            rows[e, sl] = rows[e, sl] * wb

                pltpu.async_copy(rows, acc.at[ebuf.at[s, 1]], ssem,
                                 add=True)

                @pl.when(c + 2 < mych)
                def _():
                    pltpu.make_async_copy(
                        eidx_hbm.at[mybase + c + 2], ebuf.at[s2],
                        isem).start()
                    pltpu.make_async_copy(
                        w_hbm.at[mybase + c + 2], wbuf.at[s2], isem).start()

        pltpu.make_async_copy(rowbufs[1], acc.at[ebuf.at[3, 1]], ssem).wait()

        plsc.subcore_barrier()
        pltpu.sync_copy(acc.at[pl.ds(base, nps)],
                        out_hbm.at[cid, pl.ds(base, nps)])
        if nrem:
            @pl.when(sid == NS - 1)
            def _():
                pltpu.sync_copy(acc.at[pl.ds(nps * NS, nrem)],
                                out_hbm.at[cid, pl.ds(nps * NS, nrem)])

    return spmm


def _row_block(n):
    for rb in (2000, 1000, 400, 200, 40, 8):
        if n % rb == 0:
            return rb
    return n


def _project(x, w1, w2, w3):
    n, d = x.shape
    rb = _row_block(n)

    def body(x_ref, w1_ref, w2_ref, w3_ref, t1_ref, t2_ref, t3_ref):
        xb = x_ref[...]
        t1_ref[...] = jnp.tanh(jnp.dot(xb, w1_ref[...],
                                       preferred_element_type=jnp.float32))
        t2_ref[...] = jnp.tanh(jnp.dot(xb, w2_ref[...],
                                       preferred_element_type=jnp.float32))
        t3_ref[...] = jnp.tanh(jnp.dot(xb, w3_ref[...],
                                       preferred_element_type=jnp.float32))

    out = jax.ShapeDtypeStruct((n, d), jnp.float32)
    wspec = pl.BlockSpec((d, d), lambda i: (0, 0))
    rspec = pl.BlockSpec((rb, d), lambda i: (i, 0))
    return pl.pallas_call(
        body,
        grid=(n // rb,),
        in_specs=[rspec, wspec, wspec, wspec],
        out_specs=[rspec, rspec, rspec],
        out_shape=[out, out, out],
    )(x, w1, w2, w3)


def _combine(p, bias):
    _, n, d = p.shape
    rb = _row_block(n)
    nb = n // rb

    def body(p_ref, b_ref, y_ref, m_ref):
        i = pl.program_id(0)
        y = p_ref[0] + p_ref[1] + b_ref[...]
        y_ref[...] = y
        s = jnp.sum(y, axis=0, keepdims=True)

        @pl.when(i == 0)
        def _():
            m_ref[...] = s

        @pl.when(i > 0)
        def _():
            m_ref[...] = m_ref[...] + s

        @pl.when(i == nb - 1)
        def _():
            m_ref[...] = m_ref[...] * (1.0 / n)

    return pl.pallas_call(
        body,
        grid=(nb,),
        in_specs=[pl.BlockSpec((2, rb, d), lambda i: (0, i, 0)),
                  pl.BlockSpec((1, d), lambda i: (0, 0))],
        out_specs=[pl.BlockSpec((rb, d), lambda i: (i, 0)),
                   pl.BlockSpec((1, d), lambda i: (0, 0))],
        out_shape=[jax.ShapeDtypeStruct((n, d), jnp.float32),
                   jax.ShapeDtypeStruct((1, d), jnp.float32)],
    )(p, bias)


def _final(a1, a2, a3, m, f1wt, f1b, f2wt, f2b):
    n, d = a1.shape
    rb = _row_block(n)

    def body(a1_ref, a2_ref, a3_ref, m_ref, f1w_ref, f1b_ref, f2w_ref,
             f2b_ref, out_ref, fa_ref):
        @pl.when(pl.program_id(0) == 0)
        def _():
            h = jnp.dot(m_ref[...], f1w_ref[...],
                        preferred_element_type=jnp.float32) + f1b_ref[...]
            h = jnp.maximum(h, 0.0)
            logits = jnp.dot(h, f2w_ref[...],
                             preferred_element_type=jnp.float32) + f2b_ref[...]
            mx = jnp.max(logits, axis=0, keepdims=True)
            ex = jnp.exp(logits - mx)
            fa = ex / jnp.sum(ex, axis=0, keepdims=True)
            fa_ref[...] = jnp.concatenate(
                [fa, jnp.zeros((5, d), jnp.float32)], axis=0)

        out_ref[...] = (a1_ref[...] * fa_ref[0:1] +
                        a2_ref[...] * fa_ref[1:2] +
                        a3_ref[...] * fa_ref[2:3])

    rspec = pl.BlockSpec((rb, d), lambda i: (i, 0))
    r = f1wt.shape[1]
    return pl.pallas_call(
        body,
        grid=(n // rb,),
        in_specs=[rspec, rspec, rspec,
                  pl.BlockSpec((3, d), lambda i: (0, 0)),
                  pl.BlockSpec((d, r), lambda i: (0, 0)),
                  pl.BlockSpec((1, r), lambda i: (0, 0)),
                  pl.BlockSpec((r, d), lambda i: (0, 0)),
                  pl.BlockSpec((1, d), lambda i: (0, 0))],
        out_specs=rspec,
        out_shape=jax.ShapeDtypeStruct((n, d), jnp.float32),
        scratch_shapes=[pltpu.VMEM((8, d), jnp.float32)],
    )(a1, a2, a3, m, f1wt, f1b, f2wt, f2b)


def kernel(features, edge_index, edge_weight,
           W1, b1, W2, b2, W3, b3, fc1_w, fc1_b, fc2_w, fc2_b):
    n, d = features.shape
    e = edge_weight.shape[0]
    dst = edge_index[0]
    src = edge_index[1]

    t = -(-e // C)
    ch0 = max(4, 4 * round(FRAC0 * t / (NS * 4)))
    ch1 = max(4, 4 * (-(-(t - NS * ch0) // (NS * 4))))
    ct = NS * (ch0 + ch1)
    pad = ct * C - e
    srcp = jnp.concatenate([src, jnp.zeros((pad,), jnp.int32)]
                           ).reshape(ct, 1, C)
    dstp = jnp.concatenate([dst, jnp.zeros((pad,), jnp.int32)]
                           ).reshape(ct, 1, C)
    wp = jnp.concatenate([edge_weight, jnp.zeros((pad,), jnp.float32)]
                         ).reshape(ct, C)
    eidx = jnp.concatenate([srcp, dstp], axis=1)

    spmm = _make_spmm(n, d, ch0, ch1)
    t1, t2, t3 = _project(features, W1, W2, W3)
    zb = jnp.zeros((1, d), jnp.float32)

    u1p = spmm(t1, eidx, wp)
    a1, m1 = _combine(u1p, b1.reshape(1, d))
    u2p = spmm(t2, eidx, wp)
    u2, _ = _combine(u2p, zb)
    v2p = spmm(u2, eidx, wp)
    a2, m2 = _combine(v2p, b2.reshape(1, d))
    u3p = spmm(t3, eidx, wp)
    u3, _ = _combine(u3p, zb)
    v3p = spmm(u3, eidx, wp)
    v3, _ = _combine(v3p, zb)
    w3p = spmm(v3, eidx, wp)
    a3, m3 = _combine(w3p, b3.reshape(1, d))

    m = jnp.concatenate([m1, m2, m3], axis=0)
    return _final(a1, a2, a3, m, fc1_w.T, fc1_b.reshape(1, -1),
                  fc2_w.T, fc2_b.reshape(1, d))


# speedup vs baseline: 4.4359x; 3.0373x over previous
import functools

import jax
import jax.numpy as jnp
from jax import lax
from jax.experimental import pallas as pl
from jax.experimental.pallas import tpu as pltpu
from jax.experimental.pallas import tpu_sc as plsc

NC = 2
NS = 16
NW = NC * NS
L = 16
C = 128


def _make_spmm(n, d, ch0, ch1):
    nps = 8 * (n // (8 * NS))
    nrem = n - nps * NS
    nd16 = d // L
    mesh = plsc.VectorSubcoreMesh(
        core_axis_name="c", subcore_axis_name="s", num_cores=NC,
        num_subcores=NS)

    @functools.partial(
        pl.kernel,
        out_type=jax.ShapeDtypeStruct((NC, n, d), jnp.float32),
        mesh=mesh,
        scratch_types=[
            pltpu.VMEM((4, 2, C), jnp.int32),
            pltpu.VMEM((4, C), jnp.float32),
            pltpu.VMEM((C, d), jnp.float32),
            pltpu.VMEM((C, d), jnp.float32),
            pltpu.VMEM_SHARED((n, d), jnp.float32),
            pltpu.SemaphoreType.DMA,
            pltpu.SemaphoreType.DMA,
            pltpu.SemaphoreType.DMA,
        ],
    )
    def spmm(x_hbm, eidx_hbm, w_hbm, out_hbm, ebuf, wbuf, rows0, rows1, acc,
             gsem, isem, ssem):
        cid = lax.axis_index("c")
        sid = lax.axis_index("s")
        mych = jnp.where(cid == 0, ch0, ch1)
        mybase = jnp.where(cid == 0, sid * ch0, NS * ch0 + sid * ch1)
        rowbufs = (rows0, rows1)

        @pl.loop(0, C)
        def _zero_rows(i):
            z = jnp.zeros((L,), jnp.float32)
            for j in range(nd16):
                rows0[i, pl.ds(j * L, L)] = z

        base = sid * nps
        full, rem = nps // C, nps % C
        for k in range(full):
            pltpu.sync_copy(rows0, acc.at[pl.ds(base + k * C, C)])
        if rem:
            pltpu.sync_copy(rows0.at[pl.ds(0, rem)],
                            acc.at[pl.ds(base + full * C, rem)])
        if nrem:
            @pl.when(sid == NS - 1)
            def _():
                pltpu.sync_copy(rows0.at[pl.ds(0, nrem)],
                                acc.at[pl.ds(nps * NS, nrem)])
        plsc.subcore_barrier()

        pltpu.sync_copy(eidx_hbm.at[mybase], ebuf.at[0])
        pltpu.sync_copy(w_hbm.at[mybase], wbuf.at[0])
        pltpu.make_async_copy(eidx_hbm.at[mybase + 1], ebuf.at[1],
                              isem).start()
        pltpu.make_async_copy(w_hbm.at[mybase + 1], wbuf.at[1], isem).start()
        pltpu.make_async_copy(x_hbm.at[ebuf.at[0, 0]], rows0, gsem).start()

        @pl.loop(0, mych, step=4)
        def _chunks(c0):
            for u in range(4):
                c = c0 + u
                b = u % 2
                nb = 1 - b
                s_, sp, s1, s2 = u, (u - 1) % 4, (u + 1) % 4, (u + 2) % 4
                rows = rowbufs[b]
                nxt = rowbufs[nb]
                pltpu.make_async_copy(x_hbm.at[ebuf.at[s_, 0]], rows,
                                      gsem).wait()

                @pl.when(c > 0)
                def _():
                    pltpu.make_async_copy(nxt, acc.at[ebuf.at[sp, 1]],
                                          ssem).wait()

                @pl.when(c + 1 < mych)
                def _():
                    pltpu.make_async_copy(
                        eidx_hbm.at[mybase + c + 1], ebuf.at[s1],
                        isem).wait()
                    pltpu.make_async_copy(
                        w_hbm.at[mybase + c + 1], wbuf.at[s1], isem).wait()
                    pltpu.make_async_copy(
                        x_hbm.at[ebuf.at[s1, 0]], nxt, gsem).start()

                @pl.loop(0, C // L)
                def _scale(g):
                    wv = wbuf[s_, pl.ds(g * L, L)]
                    for l in range(L):
                        wb = lax.gather(
                            wv, jnp.full((L, 1), l, jnp.int32),
                            lax.GatherDimensionNumbers(
                                offset_dims=(), collapsed_slice_dims=(0,),
                                start_index_map=(0,)),
                            slice_sizes=(1,),
                            mode=lax.GatherScatterMode.PROMISE_IN_BOUNDS)
                        e = g * L + l
                        for j in range(nd16):
                            sl = pl.ds(j * L, L)
                            rows[e, sl] = rows[e, sl] * wb

                pltpu.async_copy(rows, acc.at[ebuf.at[s_, 1]], ssem,
                                 add=True)

                @pl.when(c + 2 < mych)
                def _():
                    pltpu.make_async_copy(
                        eidx_hbm.at[mybase + c + 2], ebuf.at[s2],
                        isem).start()
                    pltpu.make_async_copy(
                        w_hbm.at[mybase + c + 2], wbuf.at[s2], isem).start()

        pltpu.make_async_copy(rowbufs[1], acc.at[ebuf.at[3, 1]], ssem).wait()

        plsc.subcore_barrier()
        pltpu.sync_copy(acc.at[pl.ds(base, nps)],
                        out_hbm.at[cid, pl.ds(base, nps)])
        if nrem:
            @pl.when(sid == NS - 1)
            def _():
                pltpu.sync_copy(acc.at[pl.ds(nps * NS, nrem)],
                                out_hbm.at[cid, pl.ds(nps * NS, nrem)])

    return spmm


def _row_block(n):
    for rb in (2000, 1000, 400, 200, 40, 8):
        if n % rb == 0:
            return rb
    return n


def _project(x, w1, w2, w3):
    n, d = x.shape
    rb = _row_block(n)

    def body(x_ref, w1_ref, w2_ref, w3_ref, t1_ref, t2_ref, t3_ref):
        xb = x_ref[...]
        for wr, tr in ((w1_ref, t1_ref), (w2_ref, t2_ref), (w3_ref, t3_ref)):
            tr[...] = jnp.tanh(jnp.dot(
                xb, wr[...], preferred_element_type=jnp.float32))

    out = jax.ShapeDtypeStruct((n, d), jnp.float32)
    wspec = pl.BlockSpec((d, d), lambda i: (0, 0))
    rspec = pl.BlockSpec((rb, d), lambda i: (i, 0))
    return pl.pallas_call(
        body,
        grid=(n // rb,),
        in_specs=[rspec, wspec, wspec, wspec],
        out_specs=[rspec, rspec, rspec],
        out_shape=[out, out, out],
    )(x, w1, w2, w3)


def _combine(p, bias):
    _, n, d = p.shape
    rb = _row_block(n)
    nb = n // rb

    def body(p_ref, b_ref, y_ref, m_ref):
        i = pl.program_id(0)
        y = p_ref[0] + p_ref[1] + b_ref[...]
        y_ref[...] = y
        s = jnp.sum(y, axis=0, keepdims=True)

        @pl.when(i == 0)
        def _():
            m_ref[...] = s

        @pl.when(i > 0)
        def _():
            m_ref[...] = m_ref[...] + s

        @pl.when(i == nb - 1)
        def _():
            m_ref[...] = m_ref[...] * (1.0 / n)

    return pl.pallas_call(
        body,
        grid=(nb,),
        in_specs=[pl.BlockSpec((2, rb, d), lambda i: (0, i, 0)),
                  pl.BlockSpec((1, d), lambda i: (0, 0))],
        out_specs=[pl.BlockSpec((rb, d), lambda i: (i, 0)),
                   pl.BlockSpec((1, d), lambda i: (0, 0))],
        out_shape=[jax.ShapeDtypeStruct((n, d), jnp.float32),
                   jax.ShapeDtypeStruct((1, d), jnp.float32)],
    )(p, bias)


def _final(a1, a2, a3, m, f1wt, f1b, f2wt, f2b):
    n, d = a1.shape
    rb = _row_block(n)

    def body(a1_ref, a2_ref, a3_ref, m_ref, f1w_ref, f1b_ref, f2w_ref,
             f2b_ref, out_ref, fa_ref):
        @pl.when(pl.program_id(0) == 0)
        def _():
            h = jnp.dot(m_ref[...], f1w_ref[...],
                        preferred_element_type=jnp.float32) + f1b_ref[...]
            h = jnp.maximum(h, 0.0)
            logits = jnp.dot(h, f2w_ref[...],
                             preferred_element_type=jnp.float32) + f2b_ref[...]
            mx = jnp.max(logits, axis=0, keepdims=True)
            ex = jnp.exp(logits - mx)
            fa = ex / jnp.sum(ex, axis=0, keepdims=True)
            fa_ref[...] = jnp.concatenate(
                [fa, jnp.zeros((5, d), jnp.float32)], axis=0)

        out_ref[...] = (a1_ref[...] * fa_ref[0:1] +
                        a2_ref[...] * fa_ref[1:2] +
                        a3_ref[...] * fa_ref[2:3])

    rspec = pl.BlockSpec((rb, d), lambda i: (i, 0))
    r = f1wt.shape[1]
    return pl.pallas_call(
        body,
        grid=(n // rb,),
        in_specs=[rspec, rspec, rspec,
                  pl.BlockSpec((3, d), lambda i: (0, 0)),
                  pl.BlockSpec((d, r), lambda i: (0, 0)),
                  pl.BlockSpec((1, r), lambda i: (0, 0)),
                  pl.BlockSpec((r, d), lambda i: (0, 0)),
                  pl.BlockSpec((1, d), lambda i: (0, 0))],
        out_specs=rspec,
        out_shape=jax.ShapeDtypeStruct((n, d), jnp.float32),
        scratch_shapes=[pltpu.VMEM((8, d), jnp.float32)],
    )(a1, a2, a3, m, f1wt, f1b, f2wt, f2b)


def kernel(features, edge_index, edge_weight,
           W1, b1, W2, b2, W3, b3, fc1_w, fc1_b, fc2_w, fc2_b):
    n, d = features.shape
    e = edge_weight.shape[0]
    dst = edge_index[0]
    src = edge_index[1]

    t = -(-e // C)
    ch0 = max(4, 4 * (-(-t // (2 * NS * 4))))
    ch1 = max(4, 4 * (-(-(t - NS * ch0) // (NS * 4))))
    ct = NS * (ch0 + ch1)
    pad = ct * C - e
    spread = (jnp.arange(pad, dtype=jnp.int32) * 37) % n
    srcp = jnp.concatenate([src, spread]).reshape(ct, 1, C)
    dstp = jnp.concatenate([dst, spread]).reshape(ct, 1, C)
    wp = jnp.concatenate([edge_weight, jnp.zeros((pad,), jnp.float32)]
                         ).reshape(ct, C)
    eidx = jnp.concatenate([srcp, dstp], axis=1)

    spmm = _make_spmm(n, d, ch0, ch1)
    t1, t2, t3 = _project(features, W1, W2, W3)
    zb = jnp.zeros((1, d), jnp.float32)

    u1p = spmm(t1, eidx, wp)
    a1, m1 = _combine(u1p, b1.reshape(1, d))
    u2p = spmm(t2, eidx, wp)
    u2, _ = _combine(u2p, zb)
    v2p = spmm(u2, eidx, wp)
    a2, m2 = _combine(v2p, b2.reshape(1, d))
    u3p = spmm(t3, eidx, wp)
    u3, _ = _combine(u3p, zb)
    v3p = spmm(u3, eidx, wp)
    v3, _ = _combine(v3p, zb)
    w3p = spmm(v3, eidx, wp)
    a3, m3 = _combine(w3p, b3.reshape(1, d))

    m = jnp.concatenate([m1, m2, m3], axis=0)
    return _final(a1, a2, a3, m, fc1_w.T, fc1_b.reshape(1, -1),
                  fc2_w.T, fc2_b.reshape(1, d))
